# baseline (speedup 1.0000x reference)
import functools

import jax
import jax.numpy as jnp
from jax import lax
from jax.experimental import pallas as pl
from jax.experimental.pallas import tpu as pltpu
from jax.experimental.pallas import tpu_sc as plsc

N_ATOMS = 100000
EMB = 128
VOCAB = 100
CHUNK = 160
NC = 2
NS = 16
NW = NC * NS
L = 16

_N_CHUNKS = N_ATOMS // CHUNK
_PER_WORKER = -(-_N_CHUNKS // NW)
_GROUPS = CHUNK // L
_UNROLL = 16


@functools.partial(
    pl.kernel,
    mesh=plsc.VectorSubcoreMesh(core_axis_name="c", subcore_axis_name="s"),
    out_type=jax.ShapeDtypeStruct((N_ATOMS, EMB), jnp.float32),
    scratch_types=[
        pltpu.VMEM((VOCAB, EMB), jnp.float32),
        pltpu.VMEM((CHUNK,), jnp.int32),
        pltpu.VMEM((CHUNK, EMB), jnp.float32),
        pltpu.VMEM((CHUNK, EMB), jnp.float32),
        pltpu.SemaphoreType.DMA,
        pltpu.SemaphoreType.DMA,
    ],
    compiler_params=pltpu.CompilerParams(needs_layout_passes=False),
)
def _emb_kernel(z_hbm, w_hbm, out_hbm, w_v, idx_v, rows_a, rows_b,
                sem_a, sem_b):
    wid = lax.axis_index("s") * NC + lax.axis_index("c")
    pltpu.sync_copy(w_hbm, w_v)
    liota = lax.iota(jnp.int32, L)

    def base_of(k):
        c = jnp.minimum(wid + k * NW, _N_CHUNKS - 1)
        return pl.multiple_of(c * CHUNK, 8)

    def compute(k, rows_v):
        base = base_of(k)
        pltpu.sync_copy(z_hbm.at[pl.ds(base, CHUNK)], idx_v)
        for g in range(_GROUPS):
            row16 = idx_v[pl.ds(g * L, L)] - 1
            atoms16 = liota + g * L

            def colstep(jo, _):
                colj = jnp.broadcast_to(jo * _UNROLL, (L,)).astype(jnp.int32)
                for _u in range(_UNROLL):
                    vals = plsc.load_gather(w_v, [row16, colj])
                    plsc.store_scatter(rows_v, [atoms16, colj], vals)
                    colj = colj + 1
                return _

            lax.fori_loop(0, EMB // _UNROLL, colstep, None)

    def scatter(k, rows_v, sem):
        pltpu.async_copy(
            rows_v, out_hbm.at[pl.ds(base_of(k), CHUNK)], sem)

    def wait(k, rows_v, sem):
        pltpu.make_async_copy(
            rows_v, out_hbm.at[pl.ds(base_of(k), CHUNK)], sem).wait()

    compute(0, rows_a)
    scatter(0, rows_a, sem_a)

    def pair(p, _):
        k = 2 * p + 1
        compute(k, rows_b)
        scatter(k, rows_b, sem_b)
        wait(k - 1, rows_a, sem_a)
        compute(k + 1, rows_a)
        scatter(k + 1, rows_a, sem_a)
        wait(k, rows_b, sem_b)
        return _

    lax.fori_loop(0, (_PER_WORKER - 2) // 2, pair, None)
    compute(_PER_WORKER - 1, rows_b)
    scatter(_PER_WORKER - 1, rows_b, sem_b)
    wait(_PER_WORKER - 2, rows_a, sem_a)
    wait(_PER_WORKER - 1, rows_b, sem_b)


def kernel(Z, W):
    return _emb_kernel(Z, W)


# speedup vs baseline: 7.6483x; 7.6483x over previous
import functools

import jax
import jax.numpy as jnp
from jax import lax
from jax.experimental import pallas as pl
from jax.experimental.pallas import tpu as pltpu
from jax.experimental.pallas import tpu_sc as plsc

N_ATOMS = 100000
EMB = 128
VOCAB = 100
CHUNK = 128
NC = 2
NS = 16
NW = NC * NS
L = 16

_N_CHUNKS = -(-N_ATOMS // CHUNK)
_LAST_BASE = N_ATOMS - CHUNK
_PER_WORKER = 25
_DEPTH = 3


@functools.partial(
    pl.kernel,
    mesh=plsc.VectorSubcoreMesh(core_axis_name="c", subcore_axis_name="s"),
    out_type=jax.ShapeDtypeStruct((N_ATOMS, EMB), jnp.float32),
    scratch_types=[
        pltpu.VMEM((_PER_WORKER, CHUNK), jnp.int32),
        [pltpu.VMEM((CHUNK, EMB), jnp.float32)] * _DEPTH,
        pltpu.SemaphoreType.DMA,
        [pltpu.SemaphoreType.DMA] * _DEPTH,
        [pltpu.SemaphoreType.DMA] * _DEPTH,
    ],
)
def _emb_kernel(z_hbm, w_hbm, out_hbm, idx_v, rows, isem, gsem, ssem):
    wid = lax.axis_index("s") * NC + lax.axis_index("c")
    woff = wid * VOCAB - 1

    def base_of(k):
        c = jnp.minimum(wid + k * NW, _N_CHUNKS - 1)
        return pl.multiple_of(jnp.minimum(c * CHUNK, _LAST_BASE), 8)

    for k in range(_PER_WORKER):
        pltpu.async_copy(z_hbm.at[pl.ds(base_of(k), CHUNK)], idx_v.at[k],
                         isem)

    def gather(k):
        b = k % _DEPTH
        pltpu.make_async_copy(z_hbm.at[pl.ds(base_of(k), CHUNK)],
                              idx_v.at[k], isem).wait()
        for j in range(CHUNK // L):
            sl = pl.ds(j * L, L)
            idx_v[k, sl] = idx_v[k, sl] + woff
        pltpu.async_copy(w_hbm.at[idx_v.at[k]], rows[b], gsem[b])

    def scatter(k):
        b = k % _DEPTH
        pltpu.make_async_copy(w_hbm.at[idx_v.at[k]], rows[b],
                              gsem[b]).wait()
        pltpu.async_copy(rows[b], out_hbm.at[pl.ds(base_of(k), CHUNK)],
                         ssem[b])

    def wait_scatter(k):
        b = k % _DEPTH
        pltpu.make_async_copy(rows[b], out_hbm.at[pl.ds(base_of(k), CHUNK)],
                              ssem[b]).wait()

    for k in range(_PER_WORKER):
        if k >= _DEPTH:
            wait_scatter(k - _DEPTH)
        gather(k)
        if k >= 1:
            scatter(k - 1)
    scatter(_PER_WORKER - 1)
    for k in range(_PER_WORKER - _DEPTH, _PER_WORKER):
        wait_scatter(k)


def kernel(Z, W):
    return _emb_kernel(Z, jnp.tile(W, (NW, 1)))


# speedup vs baseline: 11.2693x; 1.4734x over previous
import functools

import jax
import jax.numpy as jnp
from jax import lax
from jax.experimental import pallas as pl
from jax.experimental.pallas import tpu as pltpu
from jax.experimental.pallas import tpu_sc as plsc

N_ATOMS = 100000
EMB = 128
VOCAB = 100
CHUNK = 128
NC = 2
NS = 16
NW = NC * NS
L = 16

_N_CHUNKS = -(-N_ATOMS // CHUNK)
_LAST_BASE = N_ATOMS - CHUNK
_PER_WORKER = 25
_DEPTH = 3


@functools.partial(
    pl.kernel,
    mesh=plsc.VectorSubcoreMesh(core_axis_name="c", subcore_axis_name="s"),
    out_type=jax.ShapeDtypeStruct((N_ATOMS, EMB), jnp.float32),
    scratch_types=[
        pltpu.VMEM((_PER_WORKER, CHUNK), jnp.int32),
        [pltpu.VMEM((CHUNK, EMB), jnp.float32)] * _DEPTH,
        pltpu.VMEM_SHARED((VOCAB, EMB), jnp.float32),
        pltpu.SemaphoreType.DMA,
        [pltpu.SemaphoreType.DMA] * _DEPTH,
        [pltpu.SemaphoreType.DMA] * _DEPTH,
    ],
)
def _emb_kernel(z_hbm, w_hbm, out_hbm, idx_v, rows, w_sh, isem, gsem, ssem):
    wid = lax.axis_index("s") * NC + lax.axis_index("c")
    woff = -1

    @pl.when(lax.axis_index("s") == 0)
    def _():
        pltpu.sync_copy(w_hbm, w_sh)

    plsc.subcore_barrier()

    def base_of(k):
        c = jnp.minimum(wid + k * NW, _N_CHUNKS - 1)
        return pl.multiple_of(jnp.minimum(c * CHUNK, _LAST_BASE), 8)

    for k in range(_PER_WORKER):
        pltpu.async_copy(z_hbm.at[pl.ds(base_of(k), CHUNK)], idx_v.at[k],
                         isem)

    def gather(k):
        b = k % _DEPTH
        pltpu.make_async_copy(z_hbm.at[pl.ds(base_of(k), CHUNK)],
                              idx_v.at[k], isem).wait()
        for j in range(CHUNK // L):
            sl = pl.ds(j * L, L)
            idx_v[k, sl] = idx_v[k, sl] + woff
        pltpu.async_copy(w_sh.at[idx_v.at[k]], rows[b], gsem[b])

    def scatter(k):
        b = k % _DEPTH
        pltpu.make_async_copy(w_sh.at[idx_v.at[k]], rows[b],
                              gsem[b]).wait()
        pltpu.async_copy(rows[b], out_hbm.at[pl.ds(base_of(k), CHUNK)],
                         ssem[b])

    def wait_scatter(k):
        b = k % _DEPTH
        pltpu.make_async_copy(rows[b], out_hbm.at[pl.ds(base_of(k), CHUNK)],
                              ssem[b]).wait()

    for k in range(_PER_WORKER):
        if k >= _DEPTH:
            wait_scatter(k - _DEPTH)
        gather(k)
        if k >= 1:
            scatter(k - 1)
    scatter(_PER_WORKER - 1)
    for k in range(_PER_WORKER - _DEPTH, _PER_WORKER):
        wait_scatter(k)


def kernel(Z, W):
    return _emb_kernel(Z, W)
